# Initial kernel scaffold; baseline (speedup 1.0000x reference)
import jax
import jax.numpy as jnp
from jax.experimental import pallas as pl


def kernel(tokens, table):
    raise NotImplementedError("write your pallas kernel here")


# speedup vs baseline: 3.1171x; 3.1171x over previous
import functools

import jax
import jax.numpy as jnp
from jax import lax
from jax.experimental import pallas as pl
from jax.experimental.pallas import tpu as pltpu
from jax.experimental.pallas import tpu_sc as plsc

EMB = 64
SCALE = 8.0

NC = 2
NS = 16
NW = NC * NS

B = 4096 * 200
BPW = B // NW
CHUNK = 128
NCHUNK = BPW // CHUNK

_mesh = plsc.VectorSubcoreMesh(core_axis_name="c", subcore_axis_name="s")


@functools.partial(
    pl.kernel,
    mesh=_mesh,
    out_type=jax.ShapeDtypeStruct((B, EMB), jnp.float32),
    scratch_types=[
        pltpu.VMEM((BPW,), jnp.int32),
        pltpu.VMEM((CHUNK, EMB), jnp.float32),
        pltpu.SemaphoreType.DMA,
    ],
    compiler_params=pltpu.CompilerParams(use_tc_tiling_on_sc=False),
)
def _emb_lookup(tokens_hbm, table_hbm, out_hbm, idx_v, rows_v, sem):
    wid = lax.axis_index("s") * NC + lax.axis_index("c")
    base = wid * BPW
    pltpu.sync_copy(tokens_hbm.at[pl.ds(base, BPW)], idx_v)

    def chunk_body(g, _):
        off = g * CHUNK
        pltpu.async_copy(
            table_hbm.at[idx_v.at[pl.ds(off, CHUNK)]], rows_v, sem
        ).wait()

        def scale_row(r, _):
            for j in range(EMB // 16):
                sl = pl.ds(j * 16, 16)
                rows_v[r, sl] = rows_v[r, sl] * SCALE
            return 0

        lax.fori_loop(0, CHUNK, scale_row, 0)
        pltpu.sync_copy(rows_v, out_hbm.at[pl.ds(base + off, CHUNK)])
        return 0

    lax.fori_loop(0, NCHUNK, chunk_body, 0)


def kernel(tokens, table):
    out = _emb_lookup(tokens.reshape(-1), table)
    return out.reshape(tokens.shape + (EMB,))


# speedup vs baseline: 4.2516x; 1.3639x over previous
import functools

import jax
import jax.numpy as jnp
from jax import lax
from jax.experimental import pallas as pl
from jax.experimental.pallas import tpu as pltpu
from jax.experimental.pallas import tpu_sc as plsc

EMB = 64
SCALE = 8.0

NC = 2
NS = 16
NW = NC * NS

B = 4096 * 200
BPW = B // NW
CHUNK = 128
NCHUNK = BPW // CHUNK
NBUF = 4
NOUT = NCHUNK // NBUF

_mesh = plsc.VectorSubcoreMesh(core_axis_name="c", subcore_axis_name="s")


@functools.partial(
    pl.kernel,
    mesh=_mesh,
    out_type=jax.ShapeDtypeStruct((B, EMB), jnp.float32),
    scratch_types=[
        pltpu.VMEM((BPW,), jnp.int32),
        pltpu.VMEM((NBUF, CHUNK, EMB), jnp.float32),
        [pltpu.SemaphoreType.DMA] * NBUF,
        [pltpu.SemaphoreType.DMA] * NBUF,
    ],
    compiler_params=pltpu.CompilerParams(use_tc_tiling_on_sc=False),
)
def _emb_lookup(tokens_hbm, table_hbm, out_hbm, idx_v, rows_v, gsems, ssems):
    wid = lax.axis_index("s") * NC + lax.axis_index("c")
    base = wid * BPW
    pltpu.sync_copy(tokens_hbm.at[pl.ds(base, BPW)], idx_v)

    def start_gather(g, b):
        pltpu.async_copy(
            table_hbm.at[idx_v.at[pl.ds(g * CHUNK, CHUNK)]],
            rows_v.at[b],
            gsems[b],
        )

    def wait_gather(g, b):
        pltpu.make_async_copy(
            table_hbm.at[idx_v.at[pl.ds(g * CHUNK, CHUNK)]],
            rows_v.at[b],
            gsems[b],
        ).wait()

    def scale_buf(b):
        def body(r4, _):
            for k in range(4):
                for j in range(EMB // 16):
                    sl = pl.ds(j * 16, 16)
                    rows_v[b, r4 * 4 + k, sl] = rows_v[b, r4 * 4 + k, sl] * SCALE
            return 0

        lax.fori_loop(0, CHUNK // 4, body, 0)

    def scatter(g, b):
        cp = pltpu.make_async_copy(
            rows_v.at[b], out_hbm.at[pl.ds(base + g * CHUNK, CHUNK)], ssems[b]
        )
        cp.start()
        cp.wait()

    for b in range(NBUF):
        start_gather(b, b)

    def outer(i, _):
        for b in range(NBUF):
            g = i * NBUF + b
            wait_gather(g, b)
            scale_buf(b)
            scatter(g, b)
            start_gather(g + NBUF, b)
        return 0

    lax.fori_loop(0, NOUT - 1, outer, 0)

    for b in range(NBUF):
        g = (NOUT - 1) * NBUF + b
        wait_gather(g, b)
        scale_buf(b)
        scatter(g, b)


def kernel(tokens, table):
    out = _emb_lookup(tokens.reshape(-1), table)
    return out.reshape(tokens.shape + (EMB,))
